# baseline (speedup 1.0000x reference)
import jax
import jax.numpy as jnp
from jax.experimental import pallas as pl
from jax.experimental.pallas import tpu as pltpu

_BM = 400


def _support_kernel(x_ref, w_ref, out_ref):
    out_ref[...] = jnp.dot(x_ref[...], w_ref[...], preferred_element_type=jnp.float32)


def _main_kernel(support_ref, b_ref, adj_ref, out_ref):
    out = jnp.dot(adj_ref[...], support_ref[...], preferred_element_type=jnp.float32)
    out = out + b_ref[...]
    m = jnp.max(out, axis=1, keepdims=True)
    shifted = out - m
    lse = jnp.log(jnp.sum(jnp.exp(shifted), axis=1, keepdims=True))
    out_ref[...] = shifted - lse


def kernel(x, adj, W, b):
    n, nfeat = x.shape
    nclass = W.shape[1]
    b2 = b.reshape(1, nclass)

    support = pl.pallas_call(
        _support_kernel,
        out_shape=jax.ShapeDtypeStruct((n, nclass), jnp.float32),
    )(x, W)

    num_m = n // _BM
    return pl.pallas_call(
        _main_kernel,
        grid=(num_m,),
        in_specs=[
            pl.BlockSpec((n, nclass), lambda i: (0, 0)),
            pl.BlockSpec((1, nclass), lambda i: (0, 0)),
            pl.BlockSpec((_BM, n), lambda i: (i, 0)),
        ],
        out_specs=pl.BlockSpec((_BM, nclass), lambda i: (i, 0)),
        out_shape=jax.ShapeDtypeStruct((n, nclass), jnp.float32),
        compiler_params=pltpu.CompilerParams(
            dimension_semantics=("parallel",),
        ),
    )(support, b2, adj)
